# baseline (speedup 1.0000x reference)
import functools

import jax
import jax.numpy as jnp
from jax import lax
from jax.experimental import pallas as pl
from jax.experimental.pallas import tpu as pltpu
from jax.experimental.pallas import tpu_sc as plsc

_NC = 2
_NS = 16
_NW = _NC * _NS
_CHUNK = 128
_K = 2
_SUPER = _CHUNK * _K
_NBUF = 3


@functools.cache
def _build(V, D, N):
    per_w = N // _NW
    n_chunks = per_w // _CHUNK
    n_super = n_chunks // _K
    mesh = plsc.VectorSubcoreMesh(core_axis_name="c", subcore_axis_name="s")

    @functools.partial(
        pl.kernel,
        out_type=jax.ShapeDtypeStruct((N, 2 * D), jnp.float32),
        mesh=mesh,
        scratch_types=[
            pltpu.VMEM((n_chunks, _CHUNK), jnp.int32),
            pltpu.VMEM((_NBUF, _SUPER, 2 * D), jnp.float32),
            pltpu.SemaphoreType.DMA((_NBUF,)),
            pltpu.SemaphoreType.DMA((_NBUF,)),
        ],
        compiler_params=pltpu.CompilerParams(use_tc_tiling_on_sc=False),
    )
    def gather_kernel(table_hbm, idx_hbm, out_hbm, idx_v, rows_v, gsem, wsem):
        wid = lax.axis_index("s") * _NC + lax.axis_index("c")
        base = wid * per_w
        pltpu.sync_copy(idx_hbm.at[wid], idx_v)

        def issue_gathers(g, b):
            for k in range(_K):
                pltpu.async_copy(
                    table_hbm.at[idx_v.at[g * _K + k]],
                    rows_v.at[b, pl.ds(k * _CHUNK, _CHUNK)],
                    gsem.at[b],
                )

        def wait_gathers(b):
            for k in range(_K):
                pltpu.make_async_copy(
                    table_hbm.at[idx_v.at[0]],
                    rows_v.at[b, pl.ds(k * _CHUNK, _CHUNK)],
                    gsem.at[b],
                ).wait()

        def wait_write(b):
            pltpu.make_async_copy(
                rows_v.at[b, :, pl.ds(0, D)],
                out_hbm.at[pl.ds(base, _SUPER), pl.ds(0, D)],
                wsem.at[b],
            ).wait()

        for b in range(_NBUF):
            issue_gathers(b, b)

        def step(g, carry):
            b = lax.rem(g, _NBUF)
            wait_gathers(b)
            pltpu.async_copy(
                rows_v.at[b, :, pl.ds(0, D)],
                out_hbm.at[pl.ds(base + g * _SUPER, _SUPER), pl.ds(0, D)],
                wsem.at[b],
            )

            @pl.when(g + _NBUF < n_super)
            def _():
                wait_write(b)
                issue_gathers(g + _NBUF, b)

            return carry

        lax.fori_loop(0, n_super, step, 0)
        for b in range(_NBUF):
            wait_write(b)

    return gather_kernel


_R = 4096


@functools.cache
def _build_reformat(V, D):
    def body(in_ref, out_ref):
        out_ref[...] = jnp.pad(in_ref[...], ((0, 0), (0, D)))

    return pl.pallas_call(
        body,
        grid=(V // _R,),
        in_specs=[pl.BlockSpec((_R, D), lambda i: (i, 0))],
        out_specs=pl.BlockSpec((_R, 2 * D), lambda i: (i, 0)),
        out_shape=jax.ShapeDtypeStruct((V, 2 * D), jnp.float32),
    )


def kernel(x, embedding_weight):
    B, H = x.shape
    V, D = embedding_weight.shape
    N = B * H
    idx = x.reshape(_NW, N // _NW // _CHUNK, _CHUNK).astype(jnp.int32)
    t128 = _build_reformat(V, D)(embedding_weight)
    out = _build(V, D, N)(t128, idx)
    return out[:, :D].reshape(B, H, D)


# speedup vs baseline: 1.1705x; 1.1705x over previous
import functools

import jax
import jax.numpy as jnp
from jax import lax
from jax.experimental import pallas as pl
from jax.experimental.pallas import tpu as pltpu
from jax.experimental.pallas import tpu_sc as plsc

_NC = 2
_NS = 16
_NW = _NC * _NS
_CHUNK = 128
_K = 4
_SUPER = _CHUNK * _K
_NBUF = 3


@functools.cache
def _build(V, D, N):
    per_w = N // _NW
    n_chunks = per_w // _CHUNK
    n_super = n_chunks // _K
    mesh = plsc.VectorSubcoreMesh(core_axis_name="c", subcore_axis_name="s")

    @functools.partial(
        pl.kernel,
        out_type=jax.ShapeDtypeStruct((N, 2 * D), jnp.float32),
        mesh=mesh,
        scratch_types=[
            pltpu.VMEM((n_chunks, _CHUNK), jnp.int32),
            pltpu.VMEM((_NBUF, _SUPER, D), jnp.float32),
            pltpu.SemaphoreType.DMA((_NBUF,)),
            pltpu.SemaphoreType.DMA((_NBUF,)),
        ],
        compiler_params=pltpu.CompilerParams(use_tc_tiling_on_sc=False),
    )
    def gather_kernel(table_hbm, idx_hbm, out_hbm, idx_v, rows_v, gsem, wsem):
        wid = lax.axis_index("s") * _NC + lax.axis_index("c")
        base = wid * per_w
        pltpu.sync_copy(idx_hbm.at[wid], idx_v)

        def issue_gathers(g, b):
            for k in range(_K):
                pltpu.async_copy(
                    table_hbm.at[idx_v.at[g * _K + k]],
                    rows_v.at[b, pl.ds(k * _CHUNK, _CHUNK)],
                    gsem.at[b],
                )

        def wait_gathers(b):
            for k in range(_K):
                pltpu.make_async_copy(
                    table_hbm.at[idx_v.at[0]],
                    rows_v.at[b, pl.ds(k * _CHUNK, _CHUNK)],
                    gsem.at[b],
                ).wait()

        def wait_write(b):
            pltpu.make_async_copy(
                rows_v.at[b],
                out_hbm.at[pl.ds(base, _SUPER), pl.ds(0, D)],
                wsem.at[b],
            ).wait()

        for b in range(_NBUF):
            issue_gathers(b, b)

        def step(g, carry):
            b = lax.rem(g, _NBUF)
            wait_gathers(b)
            pltpu.async_copy(
                rows_v.at[b],
                out_hbm.at[pl.ds(base + g * _SUPER, _SUPER), pl.ds(0, D)],
                wsem.at[b],
            )

            @pl.when(g + _NBUF < n_super)
            def _():
                wait_write(b)
                issue_gathers(g + _NBUF, b)

            return carry

        lax.fori_loop(0, n_super, step, 0)
        for b in range(_NBUF):
            wait_write(b)

    return gather_kernel


def kernel(x, embedding_weight):
    B, H = x.shape
    V, D = embedding_weight.shape
    N = B * H
    idx = x.reshape(_NW, N // _NW // _CHUNK, _CHUNK).astype(jnp.int32)
    out = _build(V, D, N)(embedding_weight, idx)
    return out[:, :D].reshape(B, H, D)
